# Initial kernel scaffold; baseline (speedup 1.0000x reference)
import jax
import jax.numpy as jnp
from jax.experimental import pallas as pl


def kernel(predicted_quaternion, radar_input, k_matrix, translation):
    raise NotImplementedError("write your pallas kernel here")


# speedup vs baseline: 1.0056x; 1.0056x over previous
import jax
import jax.numpy as jnp
from jax.experimental import pallas as pl
from jax.experimental.pallas import tpu as pltpu

B, H, W = 8, 512, 1024
HW = H * W
ROWS = 8


def _transform_mats(predicted_quaternion, k_matrix, translation):
    q = predicted_quaternion / jnp.linalg.norm(predicted_quaternion, axis=1, keepdims=True)
    w, x, y, z = q[:, 0], q[:, 1], q[:, 2], q[:, 3]
    R = jnp.stack([
        jnp.stack([1 - 2 * (y * y + z * z), 2 * (x * y - w * z), 2 * (x * z + w * y)], axis=-1),
        jnp.stack([2 * (x * y + w * z), 1 - 2 * (x * x + z * z), 2 * (y * z - w * x)], axis=-1),
        jnp.stack([2 * (x * z - w * y), 2 * (y * z + w * x), 1 - 2 * (x * x + y * y)], axis=-1),
    ], axis=1)
    T = jnp.concatenate([R, translation[:, :, None]], axis=2)
    k_inv = jnp.linalg.inv(k_matrix)
    return T, k_inv


def _bf(v):
    u = jax.lax.bitcast_convert_type(v, jnp.uint32)
    r = (u + jnp.uint32(0x7FFF) + ((u >> 16) & jnp.uint32(1))) & jnp.uint32(0xFFFF0000)
    return jax.lax.bitcast_convert_type(r, jnp.float32)


def _two_sum(a, b):
    s = a + b
    ap = s - b
    bp = s - ap
    e = (a - ap) + (b - bp)
    return s, e


def _wide3(p0, p1, p2):
    s1, e1 = _two_sum(p0, p1)
    s2, e2 = _two_sum(s1, p2)
    return s2 + (e1 + e2)


def _wide4(p0, p1, p2, p3):
    s1, e1 = _two_sum(p0, p1)
    s2, e2 = _two_sum(s1, p2)
    s3, e3 = _two_sum(s2, p3)
    return s3 + ((e1 + e2) + e3)


def _point_kernel(coef_ref, depth_ref, flat_ref, dval_ref, cloud_ref):
    r = pl.program_id(1)
    d = depth_ref[0]
    xb = _bf(jax.lax.broadcasted_iota(jnp.int32, (ROWS, W), 1).astype(jnp.float32))
    yb = _bf((jax.lax.broadcasted_iota(jnp.int32, (ROWS, W), 0) + r * ROWS).astype(jnp.float32))

    def c(i):
        return coef_ref[0, 0, i]

    cam0 = _wide3(c(0) * xb, c(1) * yb, jnp.full((ROWS, W), c(2)))
    cam1 = _wide3(c(3) * xb, c(4) * yb, jnp.full((ROWS, W), c(5)))
    cam2 = _wide3(c(6) * xb, c(7) * yb, jnp.full((ROWS, W), c(8)))
    mask = d > 0
    cb0 = _bf(cam0 * d)
    cb1 = _bf(cam1 * d)
    cb2 = _bf(cam2 * d)
    t0 = _wide4(c(9) * cb0, c(10) * cb1, c(11) * cb2, jnp.full((ROWS, W), c(12)))
    t1 = _wide4(c(13) * cb0, c(14) * cb1, c(15) * cb2, jnp.full((ROWS, W), c(16)))
    t2 = _wide4(c(17) * cb0, c(18) * cb1, c(19) * cb2, jnp.full((ROWS, W), c(20)))
    tb0 = _bf(t0)
    tb1 = _bf(t1)
    tb2 = _bf(t2)
    p0 = _wide3(c(21) * tb0, c(22) * tb1, c(23) * tb2)
    p1 = _wide3(c(24) * tb0, c(25) * tb1, c(26) * tb2)
    zp = _wide3(c(27) * tb0, c(28) * tb1, c(29) * tb2)
    denom = jnp.where(jnp.abs(zp) > 1e-6, zp, 1.0)
    px = jnp.round(p0 / denom).astype(jnp.int32)
    py = jnp.round(p1 / denom).astype(jnp.int32)
    valid = mask & (px >= 0) & (px < W) & (py >= 0) & (py < H)
    flat_ref[0] = jnp.where(valid, py * W + px, HW)
    dval_ref[0] = jnp.where(valid, zp, 0.0)
    cloud_ref[0, 0] = jnp.where(valid, t0, 0.0)
    cloud_ref[0, 1] = jnp.where(valid, t1, 0.0)
    cloud_ref[0, 2] = jnp.where(valid, t2, 0.0)


def kernel(predicted_quaternion, radar_input, k_matrix, translation):
    T, k_inv = _transform_mats(predicted_quaternion, k_matrix, translation)

    def bfr(x):
        return jax.lax.optimization_barrier(x.astype(jnp.bfloat16)).astype(jnp.float32)

    coef = jnp.concatenate([
        bfr(k_inv).reshape(B, 9),
        bfr(T).reshape(B, 12),
        bfr(k_matrix).reshape(B, 9),
        jnp.zeros((B, 2), jnp.float32),
    ], axis=1).reshape(B, 1, 32)
    depth = radar_input.reshape(B, H, W)

    flat_idx, depth_vals, cloud = pl.pallas_call(
        _point_kernel,
        grid=(B, H // ROWS),
        in_specs=[
            pl.BlockSpec((1, 1, 32), lambda b, r: (b, 0, 0), memory_space=pltpu.SMEM),
            pl.BlockSpec((1, ROWS, W), lambda b, r: (b, r, 0)),
        ],
        out_specs=[
            pl.BlockSpec((1, ROWS, W), lambda b, r: (b, r, 0)),
            pl.BlockSpec((1, ROWS, W), lambda b, r: (b, r, 0)),
            pl.BlockSpec((1, 3, ROWS, W), lambda b, r: (b, 0, r, 0)),
        ],
        out_shape=[
            jax.ShapeDtypeStruct((B, H, W), jnp.int32),
            jax.ShapeDtypeStruct((B, H, W), jnp.float32),
            jax.ShapeDtypeStruct((B, 3, H, W), jnp.float32),
        ],
    )(coef, depth)

    flat_idx = flat_idx.reshape(B, HW)
    depth_vals = depth_vals.reshape(B, HW)
    b_idx = jnp.arange(B)[:, None]
    dm = jnp.zeros((B, HW + 1), dtype=jnp.float32).at[b_idx, flat_idx].set(depth_vals)
    depth_maps_predicted = dm[:, :HW].reshape(B, H, W)
    cloud_pred = cloud.reshape(B, 3, HW).transpose(0, 2, 1)
    return depth_maps_predicted, cloud_pred


# speedup vs baseline: 16.1089x; 16.0194x over previous
import functools

import jax
import jax.numpy as jnp
from jax import lax
from jax.experimental import pallas as pl
from jax.experimental.pallas import tpu as pltpu
from jax.experimental.pallas import tpu_sc as plsc

B, H, W = 8, 512, 1024
HW = H * W
ROWS = 8

NC, NS = 2, 16
CH = 8192
N_CHUNK = HW // CH
RG = 8
RGN = HW // RG


def _transform_mats(predicted_quaternion, k_matrix, translation):
    q = predicted_quaternion / jnp.linalg.norm(predicted_quaternion, axis=1, keepdims=True)
    w, x, y, z = q[:, 0], q[:, 1], q[:, 2], q[:, 3]
    R = jnp.stack([
        jnp.stack([1 - 2 * (y * y + z * z), 2 * (x * y - w * z), 2 * (x * z + w * y)], axis=-1),
        jnp.stack([2 * (x * y + w * z), 1 - 2 * (x * x + z * z), 2 * (y * z - w * x)], axis=-1),
        jnp.stack([2 * (x * z - w * y), 2 * (y * z + w * x), 1 - 2 * (x * x + y * y)], axis=-1),
    ], axis=1)
    T = jnp.concatenate([R, translation[:, :, None]], axis=2)
    k_inv = jnp.linalg.inv(k_matrix)
    return T, k_inv


def _bf(v):
    u = jax.lax.bitcast_convert_type(v, jnp.uint32)
    r = (u + jnp.uint32(0x7FFF) + ((u >> 16) & jnp.uint32(1))) & jnp.uint32(0xFFFF0000)
    return jax.lax.bitcast_convert_type(r, jnp.float32)


def _two_sum(a, b):
    s = a + b
    ap = s - b
    bp = s - ap
    e = (a - ap) + (b - bp)
    return s, e


def _wide3(p0, p1, p2):
    s1, e1 = _two_sum(p0, p1)
    s2, e2 = _two_sum(s1, p2)
    return s2 + (e1 + e2)


def _wide4(p0, p1, p2, p3):
    s1, e1 = _two_sum(p0, p1)
    s2, e2 = _two_sum(s1, p2)
    s3, e3 = _two_sum(s2, p3)
    return s3 + ((e1 + e2) + e3)


def _point_kernel(coef_ref, depth_ref, gidx_ref, dval_ref, cloud_ref):
    b = pl.program_id(0)
    r = pl.program_id(1)
    d = depth_ref[0]
    xi = jax.lax.broadcasted_iota(jnp.int32, (ROWS, W), 1)
    yi = jax.lax.broadcasted_iota(jnp.int32, (ROWS, W), 0) + r * ROWS
    xb = _bf(xi.astype(jnp.float32))
    yb = _bf(yi.astype(jnp.float32))

    def c(i):
        return coef_ref[0, 0, i]

    cam0 = _wide3(c(0) * xb, c(1) * yb, jnp.full((ROWS, W), c(2)))
    cam1 = _wide3(c(3) * xb, c(4) * yb, jnp.full((ROWS, W), c(5)))
    cam2 = _wide3(c(6) * xb, c(7) * yb, jnp.full((ROWS, W), c(8)))
    mask = d > 0
    cb0 = _bf(cam0 * d)
    cb1 = _bf(cam1 * d)
    cb2 = _bf(cam2 * d)
    t0 = _wide4(c(9) * cb0, c(10) * cb1, c(11) * cb2, jnp.full((ROWS, W), c(12)))
    t1 = _wide4(c(13) * cb0, c(14) * cb1, c(15) * cb2, jnp.full((ROWS, W), c(16)))
    t2 = _wide4(c(17) * cb0, c(18) * cb1, c(19) * cb2, jnp.full((ROWS, W), c(20)))
    tb0 = _bf(t0)
    tb1 = _bf(t1)
    tb2 = _bf(t2)
    p0 = _wide3(c(21) * tb0, c(22) * tb1, c(23) * tb2)
    p1 = _wide3(c(24) * tb0, c(25) * tb1, c(26) * tb2)
    zp = _wide3(c(27) * tb0, c(28) * tb1, c(29) * tb2)
    denom = jnp.where(jnp.abs(zp) > 1e-6, zp, 1.0)
    px = jnp.round(p0 / denom).astype(jnp.int32)
    py = jnp.round(p1 / denom).astype(jnp.int32)
    valid = mask & (px >= 0) & (px < W) & (py >= 0) & (py < H)
    gidx_ref[0] = jnp.where(valid, b * HW + py * W + px, -1)
    dval_ref[0] = jnp.where(valid, zp, 0.0)
    cloud_ref[0, 0] = jnp.where(valid, t0, 0.0)
    cloud_ref[0, 1] = jnp.where(valid, t1, 0.0)
    cloud_ref[0, 2] = jnp.where(valid, t2, 0.0)


_sc_mesh = plsc.VectorSubcoreMesh(core_axis_name="c", subcore_axis_name="s")


@functools.partial(
    pl.kernel,
    mesh=_sc_mesh,
    compiler_params=pltpu.CompilerParams(needs_layout_passes=False),
    out_type=jax.ShapeDtypeStruct((B * HW,), jnp.float32),
    scratch_types=[
        pltpu.VMEM((RGN,), jnp.float32),
        pltpu.VMEM((CH,), jnp.int32),
        pltpu.VMEM((CH,), jnp.int32),
        pltpu.VMEM((CH,), jnp.float32),
        pltpu.VMEM((CH,), jnp.float32),
        pltpu.SemaphoreType.DMA,
        pltpu.SemaphoreType.DMA,
        pltpu.SemaphoreType.DMA,
        pltpu.SemaphoreType.DMA,
    ],
)
def _scatter_kernel(gidx_hbm, dval_hbm, out_hbm, map_v, idx0, idx1, val0, val1,
                    si0, si1, sv0, sv1):
    cid = lax.axis_index("c")
    sid = lax.axis_index("s")
    wid = cid * NS + sid

    def scan_chunk(idx_v, val_v, tbase):
        def vbody(j, _):
            iv = idx_v[pl.ds(j * 16, 16)]
            lv = iv - tbase
            m = plsc.bitcast(lv, jnp.uint32) < jnp.uint32(RGN)
            lv2 = jnp.where(m, lv, 0)
            xv = val_v[pl.ds(j * 16, 16)]
            plsc.store_scatter(map_v, [lv2], xv, mask=m)
            return _

        lax.fori_loop(0, CH // 16, vbody, 0)

    for task in (wid, wid + 32):
        b = task // RG
        rg = task % RG
        tbase = b * HW + rg * RGN
        sbase = b * HW

        def zbody(i, _):
            map_v[pl.ds(i * 16, 16)] = jnp.zeros((16,), jnp.float32)
            return _

        lax.fori_loop(0, RGN // 16, zbody, 0)

        pltpu.async_copy(gidx_hbm.at[pl.ds(sbase, CH)], idx0, si0)
        pltpu.async_copy(dval_hbm.at[pl.ds(sbase, CH)], val0, sv0)

        def cbody(i, carry):
            k0 = i * 2
            pltpu.make_async_copy(gidx_hbm.at[pl.ds(0, CH)], idx0, si0).wait()
            pltpu.make_async_copy(dval_hbm.at[pl.ds(0, CH)], val0, sv0).wait()
            pltpu.async_copy(gidx_hbm.at[pl.ds(sbase + (k0 + 1) * CH, CH)], idx1, si1)
            pltpu.async_copy(dval_hbm.at[pl.ds(sbase + (k0 + 1) * CH, CH)], val1, sv1)
            scan_chunk(idx0, val0, tbase)

            @pl.when(i < N_CHUNK // 2 - 1)
            def _():
                pltpu.async_copy(gidx_hbm.at[pl.ds(sbase + (k0 + 2) * CH, CH)], idx0, si0)
                pltpu.async_copy(dval_hbm.at[pl.ds(sbase + (k0 + 2) * CH, CH)], val0, sv0)

            pltpu.make_async_copy(gidx_hbm.at[pl.ds(0, CH)], idx1, si1).wait()
            pltpu.make_async_copy(dval_hbm.at[pl.ds(0, CH)], val1, sv1).wait()
            scan_chunk(idx1, val1, tbase)
            return carry

        lax.fori_loop(0, N_CHUNK // 2, cbody, 0)
        pltpu.sync_copy(map_v, out_hbm.at[pl.ds(tbase, RGN)])


def kernel(predicted_quaternion, radar_input, k_matrix, translation):
    T, k_inv = _transform_mats(predicted_quaternion, k_matrix, translation)

    def bfr(x):
        return jax.lax.optimization_barrier(x.astype(jnp.bfloat16)).astype(jnp.float32)

    coef = jnp.concatenate([
        bfr(k_inv).reshape(B, 9),
        bfr(T).reshape(B, 12),
        bfr(k_matrix).reshape(B, 9),
        jnp.zeros((B, 2), jnp.float32),
    ], axis=1).reshape(B, 1, 32)
    depth = radar_input.reshape(B, H, W)

    gidx, dval, cloud = pl.pallas_call(
        _point_kernel,
        grid=(B, H // ROWS),
        in_specs=[
            pl.BlockSpec((1, 1, 32), lambda b, r: (b, 0, 0), memory_space=pltpu.SMEM),
            pl.BlockSpec((1, ROWS, W), lambda b, r: (b, r, 0)),
        ],
        out_specs=[
            pl.BlockSpec((1, ROWS, W), lambda b, r: (b, r, 0)),
            pl.BlockSpec((1, ROWS, W), lambda b, r: (b, r, 0)),
            pl.BlockSpec((1, 3, ROWS, W), lambda b, r: (b, 0, r, 0)),
        ],
        out_shape=[
            jax.ShapeDtypeStruct((B, H, W), jnp.int32),
            jax.ShapeDtypeStruct((B, H, W), jnp.float32),
            jax.ShapeDtypeStruct((B, 3, H, W), jnp.float32),
        ],
    )(coef, depth)

    dm = _scatter_kernel(gidx.reshape(B * HW), dval.reshape(B * HW))
    depth_maps_predicted = dm.reshape(B, H, W)
    cloud_pred = cloud.reshape(B, 3, HW).transpose(0, 2, 1)
    return depth_maps_predicted, cloud_pred


# speedup vs baseline: 22.4563x; 1.3940x over previous
import functools

import jax
import jax.numpy as jnp
from jax import lax
from jax.experimental import pallas as pl
from jax.experimental.pallas import tpu as pltpu
from jax.experimental.pallas import tpu_sc as plsc

B, H, W = 8, 512, 1024
HW = H * W
ROWS = 8

NC, NS = 2, 16
CH = 8192
N_CHUNK = HW // CH
RG = 8
RGN = HW // RG


def _transform_mats(predicted_quaternion, k_matrix, translation):
    q = predicted_quaternion / jnp.linalg.norm(predicted_quaternion, axis=1, keepdims=True)
    w, x, y, z = q[:, 0], q[:, 1], q[:, 2], q[:, 3]
    R = jnp.stack([
        jnp.stack([1 - 2 * (y * y + z * z), 2 * (x * y - w * z), 2 * (x * z + w * y)], axis=-1),
        jnp.stack([2 * (x * y + w * z), 1 - 2 * (x * x + z * z), 2 * (y * z - w * x)], axis=-1),
        jnp.stack([2 * (x * z - w * y), 2 * (y * z + w * x), 1 - 2 * (x * x + y * y)], axis=-1),
    ], axis=1)
    T = jnp.concatenate([R, translation[:, :, None]], axis=2)
    k_inv = jnp.linalg.inv(k_matrix)
    return T, k_inv


def _bf(v):
    u = jax.lax.bitcast_convert_type(v, jnp.uint32)
    r = (u + jnp.uint32(0x7FFF) + ((u >> 16) & jnp.uint32(1))) & jnp.uint32(0xFFFF0000)
    return jax.lax.bitcast_convert_type(r, jnp.float32)


def _two_sum(a, b):
    s = a + b
    ap = s - b
    bp = s - ap
    e = (a - ap) + (b - bp)
    return s, e


def _wide3(p0, p1, p2):
    s1, e1 = _two_sum(p0, p1)
    s2, e2 = _two_sum(s1, p2)
    return s2 + (e1 + e2)


def _wide4(p0, p1, p2, p3):
    s1, e1 = _two_sum(p0, p1)
    s2, e2 = _two_sum(s1, p2)
    s3, e3 = _two_sum(s2, p3)
    return s3 + ((e1 + e2) + e3)


def _point_kernel(coef_ref, depth_ref, gidx_ref, dval_ref, cloud_ref):
    b = pl.program_id(0)
    r = pl.program_id(1)
    d = depth_ref[0]
    xi = jax.lax.broadcasted_iota(jnp.int32, (ROWS, W), 1)
    yi = jax.lax.broadcasted_iota(jnp.int32, (ROWS, W), 0) + r * ROWS
    xb = _bf(xi.astype(jnp.float32))
    yb = _bf(yi.astype(jnp.float32))

    def c(i):
        return coef_ref[0, 0, i]

    cam0 = _wide3(c(0) * xb, c(1) * yb, jnp.full((ROWS, W), c(2)))
    cam1 = _wide3(c(3) * xb, c(4) * yb, jnp.full((ROWS, W), c(5)))
    cam2 = _wide3(c(6) * xb, c(7) * yb, jnp.full((ROWS, W), c(8)))
    mask = d > 0
    cb0 = _bf(cam0 * d)
    cb1 = _bf(cam1 * d)
    cb2 = _bf(cam2 * d)
    t0 = _wide4(c(9) * cb0, c(10) * cb1, c(11) * cb2, jnp.full((ROWS, W), c(12)))
    t1 = _wide4(c(13) * cb0, c(14) * cb1, c(15) * cb2, jnp.full((ROWS, W), c(16)))
    t2 = _wide4(c(17) * cb0, c(18) * cb1, c(19) * cb2, jnp.full((ROWS, W), c(20)))
    tb0 = _bf(t0)
    tb1 = _bf(t1)
    tb2 = _bf(t2)
    p0 = _wide3(c(21) * tb0, c(22) * tb1, c(23) * tb2)
    p1 = _wide3(c(24) * tb0, c(25) * tb1, c(26) * tb2)
    zp = _wide3(c(27) * tb0, c(28) * tb1, c(29) * tb2)
    denom = jnp.where(jnp.abs(zp) > 1e-6, zp, 1.0)
    px = jnp.round(p0 / denom).astype(jnp.int32)
    py = jnp.round(p1 / denom).astype(jnp.int32)
    valid = mask & (px >= 0) & (px < W) & (py >= 0) & (py < H)
    gidx_ref[0] = jnp.where(valid, b * HW + py * W + px, -1)
    dval_ref[0] = jnp.where(valid, zp, 0.0)
    cloud_ref[0, 0] = jnp.where(valid, t0, 0.0)
    cloud_ref[0, 1] = jnp.where(valid, t1, 0.0)
    cloud_ref[0, 2] = jnp.where(valid, t2, 0.0)


_sc_mesh = plsc.VectorSubcoreMesh(core_axis_name="c", subcore_axis_name="s")

SEG = HW // 4
N_IN = SEG // CH
FB = 2048
CBUF = 2 * FB


@functools.partial(
    pl.kernel,
    mesh=_sc_mesh,
    compiler_params=pltpu.CompilerParams(needs_layout_passes=False),
    out_type=[
        jax.ShapeDtypeStruct((B * HW,), jnp.int32),
        jax.ShapeDtypeStruct((B * HW,), jnp.float32),
        jax.ShapeDtypeStruct((NC * NS * 16,), jnp.int32),
    ],
    scratch_types=[
        pltpu.VMEM((CH,), jnp.int32),
        pltpu.VMEM((CH,), jnp.int32),
        pltpu.VMEM((CH,), jnp.float32),
        pltpu.VMEM((CH,), jnp.float32),
        pltpu.VMEM((CBUF + 16,), jnp.int32),
        pltpu.VMEM((CBUF + 16,), jnp.float32),
        pltpu.VMEM((16,), jnp.int32),
        pltpu.SemaphoreType.DMA,
        pltpu.SemaphoreType.DMA,
        pltpu.SemaphoreType.DMA,
        pltpu.SemaphoreType.DMA,
        pltpu.SemaphoreType.DMA,
        pltpu.SemaphoreType.DMA,
    ],
)
def _compact_kernel(gidx_hbm, dval_hbm, cidx_hbm, cval_hbm, cnts_hbm,
                    in_i0, in_i1, in_v0, in_v1, cb_i, cb_v, cntb,
                    si0, si1, sv0, sv1, fi, fv):
    cid = lax.axis_index("c")
    sid = lax.axis_index("s")
    wid = cid * NS + sid
    sbase = wid * SEG

    def flush(fblk):
        h = (fblk & 1) * FB
        pltpu.async_copy(cb_i.at[pl.ds(h, FB)],
                         cidx_hbm.at[pl.ds(sbase + fblk * FB, FB)], fi)
        pltpu.async_copy(cb_v.at[pl.ds(h, FB)],
                         cval_hbm.at[pl.ds(sbase + fblk * FB, FB)], fv)

    def wait_flush():
        pltpu.make_async_copy(cb_i.at[pl.ds(0, FB)],
                              cidx_hbm.at[pl.ds(0, FB)], fi).wait()
        pltpu.make_async_copy(cb_v.at[pl.ds(0, FB)],
                              cval_hbm.at[pl.ds(0, FB)], fv).wait()

    def compact_chunk(in_i, in_v, carry):
        def vbody(j, c):
            cnt, fblk = c
            iv = in_i[pl.ds(j * 16, 16)]
            xv = in_v[pl.ds(j * 16, 16)]
            m = iv >= 0
            k = jnp.sum(m.astype(jnp.int32))
            off = cnt & (CBUF - 1)
            plsc.store_compressed(cb_i.at[pl.ds(off, 16)], iv, mask=m)
            plsc.store_compressed(cb_v.at[pl.ds(off, 16)], xv, mask=m)

            @pl.when(off + k > CBUF)
            def _():
                cb_i[pl.ds(0, 16)] = cb_i[pl.ds(CBUF, 16)]
                cb_v[pl.ds(0, 16)] = cb_v[pl.ds(CBUF, 16)]

            cnt2 = cnt + k
            do_flush = cnt2 - fblk * FB >= FB

            @pl.when(do_flush)
            def _():
                @pl.when(fblk > 0)
                def _():
                    wait_flush()

                flush(fblk)

            return cnt2, jnp.where(do_flush, fblk + 1, fblk)

        return lax.fori_loop(0, CH // 16, vbody, carry)

    pltpu.async_copy(gidx_hbm.at[pl.ds(sbase, CH)], in_i0, si0)
    pltpu.async_copy(dval_hbm.at[pl.ds(sbase, CH)], in_v0, sv0)

    def cbody(i, carry):
        k0 = i * 2
        pltpu.make_async_copy(gidx_hbm.at[pl.ds(0, CH)], in_i0, si0).wait()
        pltpu.make_async_copy(dval_hbm.at[pl.ds(0, CH)], in_v0, sv0).wait()
        pltpu.async_copy(gidx_hbm.at[pl.ds(sbase + (k0 + 1) * CH, CH)], in_i1, si1)
        pltpu.async_copy(dval_hbm.at[pl.ds(sbase + (k0 + 1) * CH, CH)], in_v1, sv1)
        carry = compact_chunk(in_i0, in_v0, carry)

        @pl.when(i < N_IN // 2 - 1)
        def _():
            pltpu.async_copy(gidx_hbm.at[pl.ds(sbase + (k0 + 2) * CH, CH)], in_i0, si0)
            pltpu.async_copy(dval_hbm.at[pl.ds(sbase + (k0 + 2) * CH, CH)], in_v0, sv0)

        pltpu.make_async_copy(gidx_hbm.at[pl.ds(0, CH)], in_i1, si1).wait()
        pltpu.make_async_copy(dval_hbm.at[pl.ds(0, CH)], in_v1, sv1).wait()
        carry = compact_chunk(in_i1, in_v1, carry)
        return carry

    cnt, fblk = lax.fori_loop(0, N_IN // 2, cbody, (0, 0))

    @pl.when(fblk > 0)
    def _():
        wait_flush()

    @pl.when(cnt > fblk * FB)
    def _():
        base16 = cnt & ~15
        lane = lax.iota(jnp.int32, 16)
        plsc.store_scatter(cb_i, [(base16 & (CBUF - 1)) + lane],
                           jnp.full((16,), -1, jnp.int32),
                           mask=(base16 + lane) >= cnt)
        nfull = (fblk * FB + FB - base16 - 16) >> 4

        def sbody(j, _):
            cb_i[pl.ds((base16 + 16 + j * 16) & (CBUF - 1), 16)] = (
                jnp.full((16,), -1, jnp.int32))
            return _

        lax.fori_loop(0, nfull, sbody, 0)
        flush(fblk)
        wait_flush()

    cntb[pl.ds(0, 16)] = jnp.zeros((16,), jnp.int32) + cnt
    pltpu.sync_copy(cntb, cnts_hbm.at[pl.ds(wid * 16, 16)])


@functools.partial(
    pl.kernel,
    mesh=_sc_mesh,
    compiler_params=pltpu.CompilerParams(needs_layout_passes=False),
    out_type=jax.ShapeDtypeStruct((B * HW,), jnp.float32),
    scratch_types=[
        pltpu.VMEM((RGN,), jnp.float32),
        pltpu.VMEM((NC * NS * 16,), jnp.int32),
        pltpu.VMEM((FB,), jnp.int32),
        pltpu.VMEM((FB,), jnp.int32),
        pltpu.VMEM((FB,), jnp.float32),
        pltpu.VMEM((FB,), jnp.float32),
        pltpu.SemaphoreType.DMA,
        pltpu.SemaphoreType.DMA,
        pltpu.SemaphoreType.DMA,
        pltpu.SemaphoreType.DMA,
    ],
)
def _scatter_kernel(cidx_hbm, cval_hbm, cnts_hbm, out_hbm, map_v, cnts_v,
                    idx0, idx1, val0, val1, si0, si1, sv0, sv1):
    cid = lax.axis_index("c")
    sid = lax.axis_index("s")
    wid = cid * NS + sid
    pltpu.sync_copy(cnts_hbm, cnts_v)

    def scan_block(idx_v, val_v, tbase):
        def vbody(j, _):
            iv = idx_v[pl.ds(j * 16, 16)]
            lv = iv - tbase
            m = plsc.bitcast(lv, jnp.uint32) < jnp.uint32(RGN)
            lv2 = jnp.where(m, lv, 0)
            xv = val_v[pl.ds(j * 16, 16)]
            plsc.store_scatter(map_v, [lv2], xv, mask=m)
            return _

        lax.fori_loop(0, FB // 16, vbody, 0)

    for task in (wid, wid + 32):
        b = task // RG
        rg = task % RG
        tbase = b * HW + rg * RGN

        def zbody(i, _):
            map_v[pl.ds(i * 16, 16)] = jnp.zeros((16,), jnp.float32)
            return _

        lax.fori_loop(0, RGN // 16, zbody, 0)

        for seg in range(4):
            row = b * 4 + seg
            ci = jnp.max(cnts_v[pl.ds(row * 16, 16)])
            nblk = (ci + FB - 1) >> 11
            sbase = row * SEG

            @pl.when(nblk > 0)
            def _():
                pltpu.async_copy(cidx_hbm.at[pl.ds(sbase, FB)], idx0, si0)
                pltpu.async_copy(cval_hbm.at[pl.ds(sbase, FB)], val0, sv0)

            def pbody(i, c):
                k0 = i * 2
                pltpu.make_async_copy(cidx_hbm.at[pl.ds(0, FB)], idx0, si0).wait()
                pltpu.make_async_copy(cval_hbm.at[pl.ds(0, FB)], val0, sv0).wait()

                @pl.when(k0 + 1 < nblk)
                def _():
                    pltpu.async_copy(cidx_hbm.at[pl.ds(sbase + (k0 + 1) * FB, FB)], idx1, si1)
                    pltpu.async_copy(cval_hbm.at[pl.ds(sbase + (k0 + 1) * FB, FB)], val1, sv1)

                scan_block(idx0, val0, tbase)

                @pl.when(k0 + 1 < nblk)
                def _():
                    @pl.when(k0 + 2 < nblk)
                    def _():
                        pltpu.async_copy(cidx_hbm.at[pl.ds(sbase + (k0 + 2) * FB, FB)], idx0, si0)
                        pltpu.async_copy(cval_hbm.at[pl.ds(sbase + (k0 + 2) * FB, FB)], val0, sv0)

                    pltpu.make_async_copy(cidx_hbm.at[pl.ds(0, FB)], idx1, si1).wait()
                    pltpu.make_async_copy(cval_hbm.at[pl.ds(0, FB)], val1, sv1).wait()
                    scan_block(idx1, val1, tbase)

                return c

            lax.fori_loop(0, (nblk + 1) >> 1, pbody, 0)

        pltpu.sync_copy(map_v, out_hbm.at[pl.ds(tbase, RGN)])


def kernel(predicted_quaternion, radar_input, k_matrix, translation):
    T, k_inv = _transform_mats(predicted_quaternion, k_matrix, translation)

    def bfr(x):
        return jax.lax.optimization_barrier(x.astype(jnp.bfloat16)).astype(jnp.float32)

    coef = jnp.concatenate([
        bfr(k_inv).reshape(B, 9),
        bfr(T).reshape(B, 12),
        bfr(k_matrix).reshape(B, 9),
        jnp.zeros((B, 2), jnp.float32),
    ], axis=1).reshape(B, 1, 32)
    depth = radar_input.reshape(B, H, W)

    gidx, dval, cloud = pl.pallas_call(
        _point_kernel,
        grid=(B, H // ROWS),
        in_specs=[
            pl.BlockSpec((1, 1, 32), lambda b, r: (b, 0, 0), memory_space=pltpu.SMEM),
            pl.BlockSpec((1, ROWS, W), lambda b, r: (b, r, 0)),
        ],
        out_specs=[
            pl.BlockSpec((1, ROWS, W), lambda b, r: (b, r, 0)),
            pl.BlockSpec((1, ROWS, W), lambda b, r: (b, r, 0)),
            pl.BlockSpec((1, 3, ROWS, W), lambda b, r: (b, 0, r, 0)),
        ],
        out_shape=[
            jax.ShapeDtypeStruct((B, H, W), jnp.int32),
            jax.ShapeDtypeStruct((B, H, W), jnp.float32),
            jax.ShapeDtypeStruct((B, 3, H, W), jnp.float32),
        ],
    )(coef, depth)

    cidx, cval, cnts = _compact_kernel(gidx.reshape(B * HW), dval.reshape(B * HW))
    dm = _scatter_kernel(cidx, cval, cnts)
    depth_maps_predicted = dm.reshape(B, H, W)
    cloud_pred = cloud.reshape(B, 3, HW).transpose(0, 2, 1)
    return depth_maps_predicted, cloud_pred


# speedup vs baseline: 26.9285x; 1.1992x over previous
import functools

import jax
import jax.numpy as jnp
from jax import lax
from jax.experimental import pallas as pl
from jax.experimental.pallas import tpu as pltpu
from jax.experimental.pallas import tpu_sc as plsc

B, H, W = 8, 512, 1024
HW = H * W
ROWS = 16

NC, NS = 2, 16
CH = 8192
N_CHUNK = HW // CH
RG = 8
RGN = HW // RG


def _transform_mats(predicted_quaternion, k_matrix, translation):
    q = predicted_quaternion / jnp.linalg.norm(predicted_quaternion, axis=1, keepdims=True)
    w, x, y, z = q[:, 0], q[:, 1], q[:, 2], q[:, 3]
    R = jnp.stack([
        jnp.stack([1 - 2 * (y * y + z * z), 2 * (x * y - w * z), 2 * (x * z + w * y)], axis=-1),
        jnp.stack([2 * (x * y + w * z), 1 - 2 * (x * x + z * z), 2 * (y * z - w * x)], axis=-1),
        jnp.stack([2 * (x * z - w * y), 2 * (y * z + w * x), 1 - 2 * (x * x + y * y)], axis=-1),
    ], axis=1)
    T = jnp.concatenate([R, translation[:, :, None]], axis=2)
    k_inv = jnp.linalg.inv(k_matrix)
    return T, k_inv


def _bf(v):
    u = jax.lax.bitcast_convert_type(v, jnp.uint32)
    r = (u + jnp.uint32(0x7FFF) + ((u >> 16) & jnp.uint32(1))) & jnp.uint32(0xFFFF0000)
    return jax.lax.bitcast_convert_type(r, jnp.float32)


def _two_sum(a, b):
    s = a + b
    ap = s - b
    bp = s - ap
    e = (a - ap) + (b - bp)
    return s, e


def _wide3(p0, p1, p2):
    s1, e1 = _two_sum(p0, p1)
    s2, e2 = _two_sum(s1, p2)
    return s2 + (e1 + e2)


def _wide4(p0, p1, p2, p3):
    s1, e1 = _two_sum(p0, p1)
    s2, e2 = _two_sum(s1, p2)
    s3, e3 = _two_sum(s2, p3)
    return s3 + ((e1 + e2) + e3)


def _point_kernel(coef_ref, depth_ref, gidx_ref, dval_ref, cloud_ref):
    b = pl.program_id(0)
    r = pl.program_id(1)
    d = depth_ref[0]
    xi = jax.lax.broadcasted_iota(jnp.int32, (ROWS, W), 1)
    yi = jax.lax.broadcasted_iota(jnp.int32, (ROWS, W), 0) + r * ROWS
    xb = _bf(xi.astype(jnp.float32))
    yb = _bf(yi.astype(jnp.float32))

    def c(i):
        return coef_ref[0, 0, i]

    cam0 = _wide3(c(0) * xb, c(1) * yb, jnp.full((ROWS, W), c(2)))
    cam1 = _wide3(c(3) * xb, c(4) * yb, jnp.full((ROWS, W), c(5)))
    cam2 = _wide3(c(6) * xb, c(7) * yb, jnp.full((ROWS, W), c(8)))
    mask = d > 0
    cb0 = _bf(cam0 * d)
    cb1 = _bf(cam1 * d)
    cb2 = _bf(cam2 * d)
    t0 = _wide4(c(9) * cb0, c(10) * cb1, c(11) * cb2, jnp.full((ROWS, W), c(12)))
    t1 = _wide4(c(13) * cb0, c(14) * cb1, c(15) * cb2, jnp.full((ROWS, W), c(16)))
    t2 = _wide4(c(17) * cb0, c(18) * cb1, c(19) * cb2, jnp.full((ROWS, W), c(20)))
    tb0 = _bf(t0)
    tb1 = _bf(t1)
    tb2 = _bf(t2)
    p0 = _wide3(c(21) * tb0, c(22) * tb1, c(23) * tb2)
    p1 = _wide3(c(24) * tb0, c(25) * tb1, c(26) * tb2)
    zp = _wide3(c(27) * tb0, c(28) * tb1, c(29) * tb2)
    denom = jnp.where(jnp.abs(zp) > 1e-6, zp, 1.0)
    px = jnp.round(p0 / denom).astype(jnp.int32)
    py = jnp.round(p1 / denom).astype(jnp.int32)
    valid = mask & (px >= 0) & (px < W) & (py >= 0) & (py < H)
    gidx_ref[0] = jnp.where(valid, b * HW + py * W + px, -1)
    dval_ref[0] = jnp.where(valid, zp, 0.0)
    cloud_ref[0, 0] = jnp.where(valid, t0, 0.0)
    cloud_ref[0, 1] = jnp.where(valid, t1, 0.0)
    cloud_ref[0, 2] = jnp.where(valid, t2, 0.0)


_sc_mesh = plsc.VectorSubcoreMesh(core_axis_name="c", subcore_axis_name="s")

SEG = HW // 4
N_IN = SEG // CH
FB = 2048
CBUF = 2 * FB


@functools.partial(
    pl.kernel,
    mesh=_sc_mesh,
    compiler_params=pltpu.CompilerParams(needs_layout_passes=False),
    out_type=[
        jax.ShapeDtypeStruct((B * HW,), jnp.int32),
        jax.ShapeDtypeStruct((B * HW,), jnp.float32),
        jax.ShapeDtypeStruct((NC * NS * 16,), jnp.int32),
    ],
    scratch_types=[
        pltpu.VMEM((CH,), jnp.int32),
        pltpu.VMEM((CH,), jnp.int32),
        pltpu.VMEM((CH,), jnp.float32),
        pltpu.VMEM((CH,), jnp.float32),
        pltpu.VMEM((CBUF + 16,), jnp.int32),
        pltpu.VMEM((CBUF + 16,), jnp.float32),
        pltpu.VMEM((16,), jnp.int32),
        pltpu.SemaphoreType.DMA,
        pltpu.SemaphoreType.DMA,
        pltpu.SemaphoreType.DMA,
        pltpu.SemaphoreType.DMA,
        pltpu.SemaphoreType.DMA,
        pltpu.SemaphoreType.DMA,
    ],
)
def _compact_kernel(gidx_hbm, dval_hbm, cidx_hbm, cval_hbm, cnts_hbm,
                    in_i0, in_i1, in_v0, in_v1, cb_i, cb_v, cntb,
                    si0, si1, sv0, sv1, fi, fv):
    cid = lax.axis_index("c")
    sid = lax.axis_index("s")
    wid = cid * NS + sid
    sbase = wid * SEG

    def flush(fblk):
        h = (fblk & 1) * FB
        pltpu.async_copy(cb_i.at[pl.ds(h, FB)],
                         cidx_hbm.at[pl.ds(sbase + fblk * FB, FB)], fi)
        pltpu.async_copy(cb_v.at[pl.ds(h, FB)],
                         cval_hbm.at[pl.ds(sbase + fblk * FB, FB)], fv)

    def wait_flush():
        pltpu.make_async_copy(cb_i.at[pl.ds(0, FB)],
                              cidx_hbm.at[pl.ds(0, FB)], fi).wait()
        pltpu.make_async_copy(cb_v.at[pl.ds(0, FB)],
                              cval_hbm.at[pl.ds(0, FB)], fv).wait()

    def compact_chunk(in_i, in_v, carry):
        def vbody(j, c):
            cnt, fblk = c
            iv = in_i[pl.ds(j * 16, 16)]
            xv = in_v[pl.ds(j * 16, 16)]
            m = iv >= 0
            k = jnp.sum(m.astype(jnp.int32))
            off = cnt & (CBUF - 1)
            plsc.store_compressed(cb_i.at[pl.ds(off, 16)], iv, mask=m)
            plsc.store_compressed(cb_v.at[pl.ds(off, 16)], xv, mask=m)

            @pl.when(off + k > CBUF)
            def _():
                cb_i[pl.ds(0, 16)] = cb_i[pl.ds(CBUF, 16)]
                cb_v[pl.ds(0, 16)] = cb_v[pl.ds(CBUF, 16)]

            cnt2 = cnt + k
            do_flush = cnt2 - fblk * FB >= FB

            @pl.when(do_flush)
            def _():
                @pl.when(fblk > 0)
                def _():
                    wait_flush()

                flush(fblk)

            return cnt2, jnp.where(do_flush, fblk + 1, fblk)

        return lax.fori_loop(0, CH // 16, vbody, carry)

    pltpu.async_copy(gidx_hbm.at[pl.ds(sbase, CH)], in_i0, si0)
    pltpu.async_copy(dval_hbm.at[pl.ds(sbase, CH)], in_v0, sv0)

    def cbody(i, carry):
        k0 = i * 2
        pltpu.make_async_copy(gidx_hbm.at[pl.ds(0, CH)], in_i0, si0).wait()
        pltpu.make_async_copy(dval_hbm.at[pl.ds(0, CH)], in_v0, sv0).wait()
        pltpu.async_copy(gidx_hbm.at[pl.ds(sbase + (k0 + 1) * CH, CH)], in_i1, si1)
        pltpu.async_copy(dval_hbm.at[pl.ds(sbase + (k0 + 1) * CH, CH)], in_v1, sv1)
        carry = compact_chunk(in_i0, in_v0, carry)

        @pl.when(i < N_IN // 2 - 1)
        def _():
            pltpu.async_copy(gidx_hbm.at[pl.ds(sbase + (k0 + 2) * CH, CH)], in_i0, si0)
            pltpu.async_copy(dval_hbm.at[pl.ds(sbase + (k0 + 2) * CH, CH)], in_v0, sv0)

        pltpu.make_async_copy(gidx_hbm.at[pl.ds(0, CH)], in_i1, si1).wait()
        pltpu.make_async_copy(dval_hbm.at[pl.ds(0, CH)], in_v1, sv1).wait()
        carry = compact_chunk(in_i1, in_v1, carry)
        return carry

    cnt, fblk = lax.fori_loop(0, N_IN // 2, cbody, (0, 0))

    @pl.when(fblk > 0)
    def _():
        wait_flush()

    @pl.when(cnt > fblk * FB)
    def _():
        base16 = cnt & ~15
        lane = lax.iota(jnp.int32, 16)
        plsc.store_scatter(cb_i, [(base16 & (CBUF - 1)) + lane],
                           jnp.full((16,), -1, jnp.int32),
                           mask=(base16 + lane) >= cnt)
        nfull = (fblk * FB + FB - base16 - 16) >> 4

        def sbody(j, _):
            cb_i[pl.ds((base16 + 16 + j * 16) & (CBUF - 1), 16)] = (
                jnp.full((16,), -1, jnp.int32))
            return _

        lax.fori_loop(0, nfull, sbody, 0)
        flush(fblk)
        wait_flush()

    cntb[pl.ds(0, 16)] = jnp.zeros((16,), jnp.int32) + cnt
    pltpu.sync_copy(cntb, cnts_hbm.at[pl.ds(wid * 16, 16)])


@functools.partial(
    pl.kernel,
    mesh=_sc_mesh,
    compiler_params=pltpu.CompilerParams(needs_layout_passes=False),
    out_type=jax.ShapeDtypeStruct((B * HW,), jnp.float32),
    scratch_types=[
        pltpu.VMEM((RGN,), jnp.float32),
        pltpu.VMEM((NC * NS * 16,), jnp.int32),
        pltpu.VMEM((FB,), jnp.int32),
        pltpu.VMEM((FB,), jnp.int32),
        pltpu.VMEM((FB,), jnp.float32),
        pltpu.VMEM((FB,), jnp.float32),
        pltpu.SemaphoreType.DMA,
        pltpu.SemaphoreType.DMA,
        pltpu.SemaphoreType.DMA,
        pltpu.SemaphoreType.DMA,
    ],
)
def _scatter_kernel(cidx_hbm, cval_hbm, cnts_hbm, out_hbm, map_v, cnts_v,
                    idx0, idx1, val0, val1, si0, si1, sv0, sv1):
    cid = lax.axis_index("c")
    sid = lax.axis_index("s")
    wid = cid * NS + sid
    pltpu.sync_copy(cnts_hbm, cnts_v)

    def scan_block(idx_v, val_v, tbase):
        def vbody(j, _):
            iv = idx_v[pl.ds(j * 16, 16)]
            lv = iv - tbase
            m = plsc.bitcast(lv, jnp.uint32) < jnp.uint32(RGN)
            lv2 = jnp.where(m, lv, 0)
            xv = val_v[pl.ds(j * 16, 16)]
            plsc.store_scatter(map_v, [lv2], xv, mask=m)
            return _

        lax.fori_loop(0, FB // 16, vbody, 0)

    for task in (wid, wid + 32):
        b = task // RG
        rg = task % RG
        tbase = b * HW + rg * RGN

        def zbody(i, _):
            map_v[pl.ds(i * 16, 16)] = jnp.zeros((16,), jnp.float32)
            return _

        lax.fori_loop(0, RGN // 16, zbody, 0)

        for seg in range(4):
            row = b * 4 + seg
            ci = jnp.max(cnts_v[pl.ds(row * 16, 16)])
            nblk = (ci + FB - 1) >> 11
            sbase = row * SEG

            @pl.when(nblk > 0)
            def _():
                pltpu.async_copy(cidx_hbm.at[pl.ds(sbase, FB)], idx0, si0)
                pltpu.async_copy(cval_hbm.at[pl.ds(sbase, FB)], val0, sv0)

            def pbody(i, c):
                k0 = i * 2
                pltpu.make_async_copy(cidx_hbm.at[pl.ds(0, FB)], idx0, si0).wait()
                pltpu.make_async_copy(cval_hbm.at[pl.ds(0, FB)], val0, sv0).wait()

                @pl.when(k0 + 1 < nblk)
                def _():
                    pltpu.async_copy(cidx_hbm.at[pl.ds(sbase + (k0 + 1) * FB, FB)], idx1, si1)
                    pltpu.async_copy(cval_hbm.at[pl.ds(sbase + (k0 + 1) * FB, FB)], val1, sv1)

                scan_block(idx0, val0, tbase)

                @pl.when(k0 + 1 < nblk)
                def _():
                    @pl.when(k0 + 2 < nblk)
                    def _():
                        pltpu.async_copy(cidx_hbm.at[pl.ds(sbase + (k0 + 2) * FB, FB)], idx0, si0)
                        pltpu.async_copy(cval_hbm.at[pl.ds(sbase + (k0 + 2) * FB, FB)], val0, sv0)

                    pltpu.make_async_copy(cidx_hbm.at[pl.ds(0, FB)], idx1, si1).wait()
                    pltpu.make_async_copy(cval_hbm.at[pl.ds(0, FB)], val1, sv1).wait()
                    scan_block(idx1, val1, tbase)

                return c

            lax.fori_loop(0, (nblk + 1) >> 1, pbody, 0)

        pltpu.sync_copy(map_v, out_hbm.at[pl.ds(tbase, RGN)])


def kernel(predicted_quaternion, radar_input, k_matrix, translation):
    T, k_inv = _transform_mats(predicted_quaternion, k_matrix, translation)

    def bfr(x):
        return jax.lax.optimization_barrier(x.astype(jnp.bfloat16)).astype(jnp.float32)

    coef = jnp.concatenate([
        bfr(k_inv).reshape(B, 9),
        bfr(T).reshape(B, 12),
        bfr(k_matrix).reshape(B, 9),
        jnp.zeros((B, 2), jnp.float32),
    ], axis=1).reshape(B, 1, 32)
    depth = radar_input.reshape(B, H, W)

    gidx, dval, cloud = pl.pallas_call(
        _point_kernel,
        grid=(B, H // ROWS),
        in_specs=[
            pl.BlockSpec((1, 1, 32), lambda b, r: (b, 0, 0), memory_space=pltpu.SMEM),
            pl.BlockSpec((1, ROWS, W), lambda b, r: (b, r, 0)),
        ],
        out_specs=[
            pl.BlockSpec((1, ROWS, W), lambda b, r: (b, r, 0)),
            pl.BlockSpec((1, ROWS, W), lambda b, r: (b, r, 0)),
            pl.BlockSpec((1, 3, ROWS, W), lambda b, r: (b, 0, r, 0)),
        ],
        out_shape=[
            jax.ShapeDtypeStruct((B, H, W), jnp.int32),
            jax.ShapeDtypeStruct((B, H, W), jnp.float32),
            jax.ShapeDtypeStruct((B, 3, H, W), jnp.float32),
        ],
    )(coef, depth)

    cidx, cval, cnts = _compact_kernel(gidx.reshape(B * HW), dval.reshape(B * HW))
    dm = _scatter_kernel(cidx, cval, cnts)
    depth_maps_predicted = dm.reshape(B, H, W)
    cloud_pred = cloud.reshape(B, 3, HW).transpose(0, 2, 1)
    return depth_maps_predicted, cloud_pred


# speedup vs baseline: 29.6663x; 1.1017x over previous
import functools

import jax
import jax.numpy as jnp
from jax import lax
from jax.experimental import pallas as pl
from jax.experimental.pallas import tpu as pltpu
from jax.experimental.pallas import tpu_sc as plsc

B, H, W = 8, 512, 1024
HW = H * W
ROWS = 32

NC, NS = 2, 16
CH = 8192
N_CHUNK = HW // CH
RG = 8
RGN = HW // RG


def _transform_mats(predicted_quaternion, k_matrix, translation):
    q = predicted_quaternion / jnp.linalg.norm(predicted_quaternion, axis=1, keepdims=True)
    w, x, y, z = q[:, 0], q[:, 1], q[:, 2], q[:, 3]
    R = jnp.stack([
        jnp.stack([1 - 2 * (y * y + z * z), 2 * (x * y - w * z), 2 * (x * z + w * y)], axis=-1),
        jnp.stack([2 * (x * y + w * z), 1 - 2 * (x * x + z * z), 2 * (y * z - w * x)], axis=-1),
        jnp.stack([2 * (x * z - w * y), 2 * (y * z + w * x), 1 - 2 * (x * x + y * y)], axis=-1),
    ], axis=1)
    T = jnp.concatenate([R, translation[:, :, None]], axis=2)
    k_inv = jnp.linalg.inv(k_matrix)
    return T, k_inv


def _bf(v):
    u = jax.lax.bitcast_convert_type(v, jnp.uint32)
    r = (u + jnp.uint32(0x7FFF) + ((u >> 16) & jnp.uint32(1))) & jnp.uint32(0xFFFF0000)
    return jax.lax.bitcast_convert_type(r, jnp.float32)


def _two_sum(a, b):
    s = a + b
    ap = s - b
    bp = s - ap
    e = (a - ap) + (b - bp)
    return s, e


def _wide3(p0, p1, p2):
    s1, e1 = _two_sum(p0, p1)
    s2, e2 = _two_sum(s1, p2)
    return s2 + (e1 + e2)


def _wide4(p0, p1, p2, p3):
    s1, e1 = _two_sum(p0, p1)
    s2, e2 = _two_sum(s1, p2)
    s3, e3 = _two_sum(s2, p3)
    return s3 + ((e1 + e2) + e3)


def _point_kernel(coef_ref, depth_ref, gidx_ref, dval_ref, cloud_ref):
    b = pl.program_id(0)
    r = pl.program_id(1)
    d = depth_ref[0]
    xi = jax.lax.broadcasted_iota(jnp.int32, (ROWS, W), 1)
    yi = jax.lax.broadcasted_iota(jnp.int32, (ROWS, W), 0) + r * ROWS
    xb = _bf(xi.astype(jnp.float32))
    yb = _bf(yi.astype(jnp.float32))

    def c(i):
        return coef_ref[0, 0, i]

    cam0 = _wide3(c(0) * xb, c(1) * yb, jnp.full((ROWS, W), c(2)))
    cam1 = _wide3(c(3) * xb, c(4) * yb, jnp.full((ROWS, W), c(5)))
    cam2 = _wide3(c(6) * xb, c(7) * yb, jnp.full((ROWS, W), c(8)))
    mask = d > 0
    cb0 = _bf(cam0 * d)
    cb1 = _bf(cam1 * d)
    cb2 = _bf(cam2 * d)
    t0 = _wide4(c(9) * cb0, c(10) * cb1, c(11) * cb2, jnp.full((ROWS, W), c(12)))
    t1 = _wide4(c(13) * cb0, c(14) * cb1, c(15) * cb2, jnp.full((ROWS, W), c(16)))
    t2 = _wide4(c(17) * cb0, c(18) * cb1, c(19) * cb2, jnp.full((ROWS, W), c(20)))
    tb0 = _bf(t0)
    tb1 = _bf(t1)
    tb2 = _bf(t2)
    p0 = _wide3(c(21) * tb0, c(22) * tb1, c(23) * tb2)
    p1 = _wide3(c(24) * tb0, c(25) * tb1, c(26) * tb2)
    zp = _wide3(c(27) * tb0, c(28) * tb1, c(29) * tb2)
    denom = jnp.where(jnp.abs(zp) > 1e-6, zp, 1.0)
    px = jnp.round(p0 / denom).astype(jnp.int32)
    py = jnp.round(p1 / denom).astype(jnp.int32)
    valid = mask & (px >= 0) & (px < W) & (py >= 0) & (py < H)
    gidx_ref[0] = jnp.where(valid, b * HW + py * W + px, -1)
    dval_ref[0] = jnp.where(valid, zp, 0.0)
    cloud_ref[0, 0] = jnp.where(valid, t0, 0.0)
    cloud_ref[0, 1] = jnp.where(valid, t1, 0.0)
    cloud_ref[0, 2] = jnp.where(valid, t2, 0.0)


_sc_mesh = plsc.VectorSubcoreMesh(core_axis_name="c", subcore_axis_name="s")

SEG = HW // 4
N_IN = SEG // CH
FB = 2048
CBUF = 2 * FB


@functools.partial(
    pl.kernel,
    mesh=_sc_mesh,
    compiler_params=pltpu.CompilerParams(needs_layout_passes=False),
    out_type=[
        jax.ShapeDtypeStruct((B * HW,), jnp.int32),
        jax.ShapeDtypeStruct((B * HW,), jnp.float32),
        jax.ShapeDtypeStruct((NC * NS * 16,), jnp.int32),
    ],
    scratch_types=[
        pltpu.VMEM((CH,), jnp.int32),
        pltpu.VMEM((CH,), jnp.int32),
        pltpu.VMEM((CH,), jnp.float32),
        pltpu.VMEM((CH,), jnp.float32),
        pltpu.VMEM((CBUF + 16,), jnp.int32),
        pltpu.VMEM((CBUF + 16,), jnp.float32),
        pltpu.VMEM((16,), jnp.int32),
        pltpu.SemaphoreType.DMA,
        pltpu.SemaphoreType.DMA,
        pltpu.SemaphoreType.DMA,
        pltpu.SemaphoreType.DMA,
        pltpu.SemaphoreType.DMA,
        pltpu.SemaphoreType.DMA,
    ],
)
def _compact_kernel(gidx_hbm, dval_hbm, cidx_hbm, cval_hbm, cnts_hbm,
                    in_i0, in_i1, in_v0, in_v1, cb_i, cb_v, cntb,
                    si0, si1, sv0, sv1, fi, fv):
    cid = lax.axis_index("c")
    sid = lax.axis_index("s")
    wid = cid * NS + sid
    sbase = wid * SEG

    def flush(fblk):
        h = (fblk & 1) * FB
        pltpu.async_copy(cb_i.at[pl.ds(h, FB)],
                         cidx_hbm.at[pl.ds(sbase + fblk * FB, FB)], fi)
        pltpu.async_copy(cb_v.at[pl.ds(h, FB)],
                         cval_hbm.at[pl.ds(sbase + fblk * FB, FB)], fv)

    def wait_flush():
        pltpu.make_async_copy(cb_i.at[pl.ds(0, FB)],
                              cidx_hbm.at[pl.ds(0, FB)], fi).wait()
        pltpu.make_async_copy(cb_v.at[pl.ds(0, FB)],
                              cval_hbm.at[pl.ds(0, FB)], fv).wait()

    def compact_chunk(in_i, in_v, carry):
        def vbody(j, c):
            cnt, fblk = c
            iv = in_i[pl.ds(j * 16, 16)]
            xv = in_v[pl.ds(j * 16, 16)]
            m = iv >= 0
            k = jnp.sum(m.astype(jnp.int32))
            off = cnt & (CBUF - 1)
            plsc.store_compressed(cb_i.at[pl.ds(off, 16)], iv, mask=m)
            plsc.store_compressed(cb_v.at[pl.ds(off, 16)], xv, mask=m)

            @pl.when(off + k > CBUF)
            def _():
                cb_i[pl.ds(0, 16)] = cb_i[pl.ds(CBUF, 16)]
                cb_v[pl.ds(0, 16)] = cb_v[pl.ds(CBUF, 16)]

            cnt2 = cnt + k
            do_flush = cnt2 - fblk * FB >= FB

            @pl.when(do_flush)
            def _():
                @pl.when(fblk > 0)
                def _():
                    wait_flush()

                flush(fblk)

            return cnt2, jnp.where(do_flush, fblk + 1, fblk)

        return lax.fori_loop(0, CH // 16, vbody, carry)

    pltpu.async_copy(gidx_hbm.at[pl.ds(sbase, CH)], in_i0, si0)
    pltpu.async_copy(dval_hbm.at[pl.ds(sbase, CH)], in_v0, sv0)

    def cbody(i, carry):
        k0 = i * 2
        pltpu.make_async_copy(gidx_hbm.at[pl.ds(0, CH)], in_i0, si0).wait()
        pltpu.make_async_copy(dval_hbm.at[pl.ds(0, CH)], in_v0, sv0).wait()
        pltpu.async_copy(gidx_hbm.at[pl.ds(sbase + (k0 + 1) * CH, CH)], in_i1, si1)
        pltpu.async_copy(dval_hbm.at[pl.ds(sbase + (k0 + 1) * CH, CH)], in_v1, sv1)
        carry = compact_chunk(in_i0, in_v0, carry)

        @pl.when(i < N_IN // 2 - 1)
        def _():
            pltpu.async_copy(gidx_hbm.at[pl.ds(sbase + (k0 + 2) * CH, CH)], in_i0, si0)
            pltpu.async_copy(dval_hbm.at[pl.ds(sbase + (k0 + 2) * CH, CH)], in_v0, sv0)

        pltpu.make_async_copy(gidx_hbm.at[pl.ds(0, CH)], in_i1, si1).wait()
        pltpu.make_async_copy(dval_hbm.at[pl.ds(0, CH)], in_v1, sv1).wait()
        carry = compact_chunk(in_i1, in_v1, carry)
        return carry

    cnt, fblk = lax.fori_loop(0, N_IN // 2, cbody, (0, 0))

    @pl.when(fblk > 0)
    def _():
        wait_flush()

    @pl.when(cnt > fblk * FB)
    def _():
        base16 = cnt & ~15
        lane = lax.iota(jnp.int32, 16)
        plsc.store_scatter(cb_i, [(base16 & (CBUF - 1)) + lane],
                           jnp.full((16,), -1, jnp.int32),
                           mask=(base16 + lane) >= cnt)
        nfull = (fblk * FB + FB - base16 - 16) >> 4

        def sbody(j, _):
            cb_i[pl.ds((base16 + 16 + j * 16) & (CBUF - 1), 16)] = (
                jnp.full((16,), -1, jnp.int32))
            return _

        lax.fori_loop(0, nfull, sbody, 0)
        flush(fblk)
        wait_flush()

    cntb[pl.ds(0, 16)] = jnp.zeros((16,), jnp.int32) + cnt
    pltpu.sync_copy(cntb, cnts_hbm.at[pl.ds(wid * 16, 16)])


@functools.partial(
    pl.kernel,
    mesh=_sc_mesh,
    compiler_params=pltpu.CompilerParams(needs_layout_passes=False),
    out_type=jax.ShapeDtypeStruct((B * HW,), jnp.float32),
    scratch_types=[
        pltpu.VMEM((RGN,), jnp.float32),
        pltpu.VMEM((NC * NS * 16,), jnp.int32),
        pltpu.VMEM((FB,), jnp.int32),
        pltpu.VMEM((FB,), jnp.int32),
        pltpu.VMEM((FB,), jnp.float32),
        pltpu.VMEM((FB,), jnp.float32),
        pltpu.SemaphoreType.DMA,
        pltpu.SemaphoreType.DMA,
        pltpu.SemaphoreType.DMA,
        pltpu.SemaphoreType.DMA,
    ],
)
def _scatter_kernel(cidx_hbm, cval_hbm, cnts_hbm, out_hbm, map_v, cnts_v,
                    idx0, idx1, val0, val1, si0, si1, sv0, sv1):
    cid = lax.axis_index("c")
    sid = lax.axis_index("s")
    wid = cid * NS + sid
    pltpu.sync_copy(cnts_hbm, cnts_v)

    def scan_block(idx_v, val_v, tbase):
        def vbody(j, _):
            iv = idx_v[pl.ds(j * 16, 16)]
            lv = iv - tbase
            m = plsc.bitcast(lv, jnp.uint32) < jnp.uint32(RGN)
            lv2 = jnp.where(m, lv, 0)
            xv = val_v[pl.ds(j * 16, 16)]
            plsc.store_scatter(map_v, [lv2], xv, mask=m)
            return _

        lax.fori_loop(0, FB // 16, vbody, 0)

    for task in (wid, wid + 32):
        b = task // RG
        rg = task % RG
        tbase = b * HW + rg * RGN

        def zbody(i, _):
            map_v[pl.ds(i * 16, 16)] = jnp.zeros((16,), jnp.float32)
            return _

        lax.fori_loop(0, RGN // 16, zbody, 0)

        for seg in range(4):
            row = b * 4 + seg
            ci = jnp.max(cnts_v[pl.ds(row * 16, 16)])
            nblk = (ci + FB - 1) >> 11
            sbase = row * SEG

            @pl.when(nblk > 0)
            def _():
                pltpu.async_copy(cidx_hbm.at[pl.ds(sbase, FB)], idx0, si0)
                pltpu.async_copy(cval_hbm.at[pl.ds(sbase, FB)], val0, sv0)

            def pbody(i, c):
                k0 = i * 2
                pltpu.make_async_copy(cidx_hbm.at[pl.ds(0, FB)], idx0, si0).wait()
                pltpu.make_async_copy(cval_hbm.at[pl.ds(0, FB)], val0, sv0).wait()

                @pl.when(k0 + 1 < nblk)
                def _():
                    pltpu.async_copy(cidx_hbm.at[pl.ds(sbase + (k0 + 1) * FB, FB)], idx1, si1)
                    pltpu.async_copy(cval_hbm.at[pl.ds(sbase + (k0 + 1) * FB, FB)], val1, sv1)

                scan_block(idx0, val0, tbase)

                @pl.when(k0 + 1 < nblk)
                def _():
                    @pl.when(k0 + 2 < nblk)
                    def _():
                        pltpu.async_copy(cidx_hbm.at[pl.ds(sbase + (k0 + 2) * FB, FB)], idx0, si0)
                        pltpu.async_copy(cval_hbm.at[pl.ds(sbase + (k0 + 2) * FB, FB)], val0, sv0)

                    pltpu.make_async_copy(cidx_hbm.at[pl.ds(0, FB)], idx1, si1).wait()
                    pltpu.make_async_copy(cval_hbm.at[pl.ds(0, FB)], val1, sv1).wait()
                    scan_block(idx1, val1, tbase)

                return c

            lax.fori_loop(0, (nblk + 1) >> 1, pbody, 0)

        pltpu.sync_copy(map_v, out_hbm.at[pl.ds(tbase, RGN)])


def kernel(predicted_quaternion, radar_input, k_matrix, translation):
    T, k_inv = _transform_mats(predicted_quaternion, k_matrix, translation)

    def bfr(x):
        return jax.lax.optimization_barrier(x.astype(jnp.bfloat16)).astype(jnp.float32)

    coef = jnp.concatenate([
        bfr(k_inv).reshape(B, 9),
        bfr(T).reshape(B, 12),
        bfr(k_matrix).reshape(B, 9),
        jnp.zeros((B, 2), jnp.float32),
    ], axis=1).reshape(B, 1, 32)
    depth = radar_input.reshape(B, H, W)

    gidx, dval, cloud = pl.pallas_call(
        _point_kernel,
        grid=(B, H // ROWS),
        in_specs=[
            pl.BlockSpec((1, 1, 32), lambda b, r: (b, 0, 0), memory_space=pltpu.SMEM),
            pl.BlockSpec((1, ROWS, W), lambda b, r: (b, r, 0)),
        ],
        out_specs=[
            pl.BlockSpec((1, ROWS, W), lambda b, r: (b, r, 0)),
            pl.BlockSpec((1, ROWS, W), lambda b, r: (b, r, 0)),
            pl.BlockSpec((1, 3, ROWS, W), lambda b, r: (b, 0, r, 0)),
        ],
        out_shape=[
            jax.ShapeDtypeStruct((B, H, W), jnp.int32),
            jax.ShapeDtypeStruct((B, H, W), jnp.float32),
            jax.ShapeDtypeStruct((B, 3, H, W), jnp.float32),
        ],
    )(coef, depth)

    cidx, cval, cnts = _compact_kernel(gidx.reshape(B * HW), dval.reshape(B * HW))
    dm = _scatter_kernel(cidx, cval, cnts)
    depth_maps_predicted = dm.reshape(B, H, W)
    cloud_pred = cloud.reshape(B, 3, HW).transpose(0, 2, 1)
    return depth_maps_predicted, cloud_pred
